# baseline (speedup 1.0000x reference)
import functools

import jax
import jax.numpy as jnp
from jax.experimental import pallas as pl

_KS0 = 0
_KS1 = 42
_KS2 = _KS0 ^ _KS1 ^ 0x1BD11BDA

_ROT_A = (13, 15, 26, 6)
_ROT_B = (17, 29, 16, 24)


def _rotl(x, r):
    return (x << jnp.uint32(r)) | (x >> jnp.uint32(32 - r))


def _threefry_bits(n_u32):
    ks = (jnp.uint32(_KS0), jnp.uint32(_KS1), jnp.uint32(_KS2))
    x0 = jnp.full_like(n_u32, jnp.uint32(_KS0))
    x1 = n_u32 + jnp.uint32(_KS1)
    for i in range(5):
        rots = _ROT_A if i % 2 == 0 else _ROT_B
        for r in rots:
            x0 = x0 + x1
            x1 = _rotl(x1, r)
            x1 = x1 ^ x0
        x0 = x0 + ks[(i + 1) % 3]
        x1 = x1 + ks[(i + 2) % 3] + jnp.uint32(i + 1)
    return x0 ^ x1


def _sort_body(labels_ref, out_ref, info_ref, c_ref, w_ref):
    labels = labels_ref[...]
    e = (info_ref[...] - labels) ** 2
    d = (out_ref[...] - labels) ** 2
    rows, lanes = e.shape
    n = rows * lanes

    key = jax.lax.bitcast_convert_type(e, jnp.uint32)
    row_id = jax.lax.broadcasted_iota(jnp.int32, (rows, lanes), 0)
    lane_id = jax.lax.broadcasted_iota(jnp.int32, (rows, lanes), 1)
    idx = row_id * lanes + lane_id

    def exchange(x, s):
        if s < lanes:
            up = jnp.roll(x, -s, axis=1)
            dn = jnp.roll(x, s, axis=1)
            mask = (lane_id & s) == 0
        else:
            rs = s // lanes
            up = jnp.roll(x, -rs, axis=0)
            dn = jnp.roll(x, rs, axis=0)
            mask = (row_id & rs) == 0
        return jnp.where(mask, up, dn)

    k = 2
    while k <= n:
        s = k // 2
        while s >= 1:
            if s < lanes:
                lower = (lane_id & s) == 0
            else:
                lower = (row_id & (s // lanes)) == 0
            if k < lanes:
                asc = (lane_id & k) == 0
            elif k < n:
                asc = (row_id & (k // lanes)) == 0
            else:
                asc = jnp.full((rows, lanes), True)
            key_p = exchange(key, s)
            idx_p = exchange(idx, s)
            d_p = exchange(d, s)
            take_min = asc == lower
            self_first = (key < key_p) | ((key == key_p) & (idx < idx_p))
            keep_self = self_first == take_min
            key = jnp.where(keep_self, key, key_p)
            idx = jnp.where(keep_self, idx, idx_p)
            d = jnp.where(keep_self, d, d_p)
            s //= 2
        k *= 2

    c_ref[...] = jnp.log((idx + 1).astype(jnp.float32))
    w_ref[...] = d


def _gumbel_body(c_ref, w_ref, loss_ref, *, rows_per_step, chunk, bs):
    step = pl.program_id(0)
    n_steps = pl.num_programs(0)
    n_chunks = bs // chunk
    row0 = step * rows_per_step

    row_iota = jax.lax.broadcasted_iota(jnp.int32, (rows_per_step, chunk), 0)
    col_iota = jax.lax.broadcasted_iota(jnp.int32, (rows_per_step, chunk), 1)
    tiny = jnp.float32(jnp.finfo(jnp.float32).tiny)

    def chunk_step(t, carry):
        acc_y, acc_w = carry
        col0 = t * chunk
        n = (row0 + row_iota) * bs + (col0 + col_iota)
        bits = _threefry_bits(n.astype(jnp.uint32))
        fb = (bits >> jnp.uint32(9)) | jnp.uint32(0x3F800000)
        f = jax.lax.bitcast_convert_type(fb, jnp.float32) - jnp.float32(1.0)
        u = f + tiny
        g = -jnp.log(-jnp.log(u))
        y = g + c_ref[pl.ds(t, 1), :]
        wv = jnp.broadcast_to(w_ref[pl.ds(t, 1), :], y.shape)
        upd = y > acc_y
        return jnp.where(upd, y, acc_y), jnp.where(upd, wv, acc_w)

    acc_y = jnp.full((rows_per_step, chunk), -jnp.inf, dtype=jnp.float32)
    acc_w = jnp.zeros((rows_per_step, chunk), dtype=jnp.float32)
    acc_y, acc_w = jax.lax.fori_loop(0, n_chunks, chunk_step, (acc_y, acc_w))

    m = jnp.max(acc_y, axis=1, keepdims=True)
    pay = jnp.max(jnp.where(acc_y == m, acc_w, jnp.float32(-1.0)), axis=1)
    part = jnp.sum(pay).reshape(1, 1)

    @pl.when(step == 0)
    def _():
        loss_ref[...] = jnp.zeros((1, 1), jnp.float32)

    loss_ref[...] += part

    @pl.when(step == n_steps - 1)
    def _():
        loss_ref[...] = loss_ref[...] / jnp.float32(bs)


@jax.jit
def kernel(i, labels, out, info):
    del i
    bs = labels.shape[0]
    lanes = 128
    rows = bs // lanes
    shape2d = (rows, lanes)

    c, w = pl.pallas_call(
        _sort_body,
        out_shape=(
            jax.ShapeDtypeStruct(shape2d, jnp.float32),
            jax.ShapeDtypeStruct(shape2d, jnp.float32),
        ),
    )(labels.reshape(shape2d), out.reshape(shape2d), info.reshape(shape2d))

    chunk = min(2048, bs)
    n_chunks = bs // chunk
    rows_per_step = 8
    grid = (bs // rows_per_step,)

    c = c.reshape(n_chunks, chunk)
    w = w.reshape(n_chunks, chunk)

    loss = pl.pallas_call(
        functools.partial(
            _gumbel_body, rows_per_step=rows_per_step, chunk=chunk, bs=bs),
        grid=grid,
        in_specs=[
            pl.BlockSpec((n_chunks, chunk), lambda s: (0, 0)),
            pl.BlockSpec((n_chunks, chunk), lambda s: (0, 0)),
        ],
        out_specs=pl.BlockSpec((1, 1), lambda s: (0, 0)),
        out_shape=jax.ShapeDtypeStruct((1, 1), jnp.float32),
    )(c, w)

    return loss.reshape(())
